# baseline (device time: 2954436 ns/iter reference)
import jax
import jax.numpy as jnp
from jax import lax
from jax.experimental import pallas as pl
from jax.experimental.pallas import tpu as pltpu

M = 8192
N = 4096
G = 4
GC = N // G
TM = 512

_ROTS = ((0, 1, 2), (2, 0, 1), (1, 2, 0))
_ALLOC = (5, 2, 1, 1, 2, 5)


def kernel(x, w_mat):
    K = x.shape[1]

    def body(x_hbm, w_ref, out_hbm, recv1, recv2, recv3,
             x_vmem, gemm_vmem, acc_vmem, add_vmem,
             local_sems, send_sems, recv_sems):
        my = lax.axis_index("i")
        z = my // 4
        j = my % 4
        y = j // 2
        xb = ((j + 1) // 2) % 2
        pz = my ^ 4
        py = 4 * z + (3 - j)
        px = my ^ 1

        bsem = pltpu.get_barrier_semaphore()
        for p in (px, py, pz):
            pl.semaphore_signal(bsem, inc=1, device_id=(p,),
                                device_id_type=pl.DeviceIdType.MESH)
        pl.semaphore_wait(bsem, 3)

        axes = ((pz, z), (py, y), (px, xb))

        def gemm_tile(g, t):
            ld = pltpu.make_async_copy(
                x_hbm.at[pl.ds(t * TM, TM)], x_vmem, local_sems.at[0])
            ld.start()
            ld.wait()
            gemm_vmem[...] = jnp.dot(
                x_vmem[...], w_ref[:, g * GC:(g + 1) * GC],
                preferred_element_type=jnp.float32)
            st = pltpu.make_async_copy(
                gemm_vmem,
                out_hbm.at[pl.ds(t * TM, TM), pl.ds(g * GC, GC)],
                local_sems.at[1])
            st.start()
            st.wait()

        def add_from(recv_buf, keep_base, nrows, g, fuse_silu):
            def step(i, carry):
                r0 = keep_base + i * TM
                c1 = pltpu.make_async_copy(
                    out_hbm.at[pl.ds(r0, TM), pl.ds(g * GC, GC)],
                    acc_vmem, local_sems.at[0])
                c2 = pltpu.make_async_copy(
                    recv_buf.at[pl.ds(i * TM, TM)], add_vmem,
                    local_sems.at[1])
                c1.start()
                c2.start()
                c1.wait()
                c2.wait()
                s = acc_vmem[...] + add_vmem[...]
                if fuse_silu:
                    s = s * jax.nn.sigmoid(s)
                acc_vmem[...] = s
                st = pltpu.make_async_copy(
                    acc_vmem,
                    out_hbm.at[pl.ds(r0, TM), pl.ds(g * GC, GC)],
                    local_sems.at[0])
                st.start()
                st.wait()
                return carry

            lax.fori_loop(0, nrows // TM, step, 0)

        def run_chain(g, next_tiles):
            rot = _ROTS[g % 3]
            (p1, b1bit) = axes[rot[0]]
            (p2, b2bit) = axes[rot[1]]
            (p3, b3bit) = axes[rot[2]]
            k1 = b1bit * 4096
            k2 = k1 + b2bit * 2048
            k3 = k2 + b3bit * 1024
            s1 = (1 - b1bit) * 4096
            s2 = k1 + (1 - b2bit) * 2048
            s3 = k2 + (1 - b3bit) * 1024
            cs = pl.ds(g * GC, GC)

            stages = (
                (p1, s1, 4096, recv1, 0, k1),
                (p2, s2, 2048, recv2, 1, k2),
                (p3, s3, 1024, recv3, 2, k3),
                (p3, k3, 1024, None, 3, None),
                (p2, k2, 2048, None, 4, None),
                (p1, k1, 4096, None, 5, None),
            )
            ti = 0
            for sidx, (prt, sb, nr, rbuf, sem, kb) in enumerate(stages):
                src = out_hbm.at[pl.ds(sb, nr), cs]
                dst = rbuf if rbuf is not None else src
                rdma = pltpu.make_async_remote_copy(
                    src_ref=src,
                    dst_ref=dst,
                    send_sem=send_sems.at[sem],
                    recv_sem=recv_sems.at[sem],
                    device_id=(prt,),
                    device_id_type=pl.DeviceIdType.MESH,
                )
                rdma.start()
                for _ in range(_ALLOC[sidx]):
                    if ti < len(next_tiles):
                        gemm_tile(*next_tiles[ti])
                        ti += 1
                rdma.wait()
                if rbuf is not None:
                    add_from(rbuf, kb, nr, g, fuse_silu=(sidx == 2))
            while ti < len(next_tiles):
                gemm_tile(*next_tiles[ti])
                ti += 1

        n_tiles = M // TM
        for t in range(n_tiles):
            gemm_tile(0, t)
        for g in range(G):
            nxt = [(g + 1, t) for t in range(n_tiles)] if g + 1 < G else []
            run_chain(g, nxt)

    outs = pl.pallas_call(
        body,
        out_shape=(
            jax.ShapeDtypeStruct((M, N), jnp.float32),
            jax.ShapeDtypeStruct((4096, GC), jnp.float32),
            jax.ShapeDtypeStruct((2048, GC), jnp.float32),
            jax.ShapeDtypeStruct((1024, GC), jnp.float32),
        ),
        in_specs=[
            pl.BlockSpec(memory_space=pl.ANY),
            pl.BlockSpec(memory_space=pltpu.VMEM),
        ],
        out_specs=(pl.BlockSpec(memory_space=pl.ANY),) * 4,
        scratch_shapes=[
            pltpu.VMEM((TM, K), jnp.float32),
            pltpu.VMEM((TM, GC), jnp.float32),
            pltpu.VMEM((TM, GC), jnp.float32),
            pltpu.VMEM((TM, GC), jnp.float32),
            pltpu.SemaphoreType.DMA((2,)),
            pltpu.SemaphoreType.DMA((6,)),
            pltpu.SemaphoreType.DMA((6,)),
        ],
        compiler_params=pltpu.CompilerParams(
            collective_id=0, vmem_limit_bytes=60 * 1024 * 1024),
    )(x, w_mat)
    return outs[0]


# device time: 1188261 ns/iter; 2.4864x vs baseline; 2.4864x over previous
import jax
import jax.numpy as jnp
from jax import lax
from jax.experimental import pallas as pl
from jax.experimental.pallas import tpu as pltpu

M = 8192
N = 4096
TM = 512
WIDTHS = (1408, 1408, 1280)
C0S = (0, 1408, 2816)
ROTS = ((0, 1, 2), (2, 0, 1), (1, 2, 0))
WMAX = 1408


def kernel(x, w_mat):
    K = x.shape[1]

    def body(x_hbm, w_ref, out_hbm, recv1, recv2, recv3,
             x_vmem, gemm_vmem, acc_vmem, add_vmem,
             local_sems, ssems, rsems):
        my = lax.axis_index("i")
        z = my // 4
        j = my % 4
        y = j // 2
        xb = ((j + 1) // 2) % 2
        pz = my ^ 4
        py = 4 * z + (3 - j)
        px = my ^ 1
        axes = ((pz, z), (py, y), (px, xb))

        bsem = pltpu.get_barrier_semaphore()
        for p in (px, py, pz):
            pl.semaphore_signal(bsem, inc=1, device_id=(p,),
                                device_id_type=pl.DeviceIdType.MESH)
        pl.semaphore_wait(bsem, 3)

        grp = []
        for g in range(3):
            w, c0 = WIDTHS[g], C0S[g]
            (p1, b1), (p2, b2), (p3, b3) = (axes[a] for a in ROTS[g])
            k1 = b1 * 4096
            k2 = k1 + b2 * 2048
            k3 = k2 + b3 * 1024
            s1 = (1 - b1) * 4096
            s2 = k1 + (1 - b2) * 2048
            s3 = k2 + (1 - b3) * 1024
            grp.append(dict(w=w, c0=c0, p=(p1, p2, p3), b=(b1, b2, b3),
                            k=(k1, k2, k3), s=(s1, s2, s3)))

        def cs(g):
            return pl.ds(grp[g]["c0"], grp[g]["w"])

        def gemm_rows(g, base, ntiles):
            w, c0 = grp[g]["w"], grp[g]["c0"]

            def step(i, c):
                r0 = base + i * TM
                ld = pltpu.make_async_copy(
                    x_hbm.at[pl.ds(r0, TM)], x_vmem, local_sems.at[0])
                ld.start()
                ld.wait()
                gemm_vmem[:, 0:w] = jnp.dot(
                    x_vmem[...], w_ref[:, c0:c0 + w],
                    preferred_element_type=jnp.float32)
                st = pltpu.make_async_copy(
                    gemm_vmem.at[:, pl.ds(0, w)],
                    out_hbm.at[pl.ds(r0, TM), cs(g)], local_sems.at[1])
                st.start()
                st.wait()
                return c

            lax.fori_loop(0, ntiles, step, 0)

        def add_rows(g, recv_buf, recv_r0, out_r0, nrows, fuse_silu):
            w = grp[g]["w"]

            def step(i, c):
                a = pltpu.make_async_copy(
                    out_hbm.at[pl.ds(out_r0 + i * TM, TM), cs(g)],
                    acc_vmem.at[:, pl.ds(0, w)], local_sems.at[0])
                b = pltpu.make_async_copy(
                    recv_buf.at[pl.ds(recv_r0 + i * TM, TM), cs(g)],
                    add_vmem.at[:, pl.ds(0, w)], local_sems.at[1])
                a.start()
                b.start()
                a.wait()
                b.wait()
                s = acc_vmem[:, 0:w] + add_vmem[:, 0:w]
                if fuse_silu:
                    s = s * jax.nn.sigmoid(s)
                acc_vmem[:, 0:w] = s
                st = pltpu.make_async_copy(
                    acc_vmem.at[:, pl.ds(0, w)],
                    out_hbm.at[pl.ds(out_r0 + i * TM, TM), cs(g)],
                    local_sems.at[0])
                st.start()
                st.wait()
                return c

            lax.fori_loop(0, nrows // TM, step, 0)

        def start_stage(g, si, src_base, nrows, recv_buf):
            src = out_hbm.at[pl.ds(src_base, nrows), cs(g)]
            dst = (recv_buf.at[:, cs(g)] if recv_buf is not None else src)
            r = pltpu.make_async_remote_copy(
                src_ref=src, dst_ref=dst,
                send_sem=ssems.at[g, si], recv_sem=rsems.at[g, si],
                device_id=(grp[g]["p"][si % 3 if si < 3 else 5 - si],),
                device_id_type=pl.DeviceIdType.MESH,
            )
            r.start()
            return r

        rd = [None, None, None]
        for g in range(3):
            gemm_rows(g, grp[g]["s"][0], 8)
            rd[g] = start_stage(g, 0, grp[g]["s"][0], 4096, recv1)
        for g in range(3):
            gemm_rows(g, grp[g]["k"][0], 8)
        for g in range(3):
            rd[g].wait()
        for g in range(3):
            add_rows(g, recv1, grp[g]["s"][1] - grp[g]["k"][0],
                     grp[g]["s"][1], 2048, False)

        for g in range(3):
            rd[g] = start_stage(g, 1, grp[g]["s"][1], 2048, recv2)
        for g in range(3):
            add_rows(g, recv1, grp[g]["k"][1] - grp[g]["k"][0],
                     grp[g]["k"][1], 2048, False)
        for g in range(3):
            rd[g].wait()
        for g in range(3):
            add_rows(g, recv2, grp[g]["s"][2] - grp[g]["k"][1],
                     grp[g]["s"][2], 1024, False)

        for g in range(3):
            rd[g] = start_stage(g, 2, grp[g]["s"][2], 1024, recv3)
        for g in range(3):
            add_rows(g, recv2, grp[g]["k"][2] - grp[g]["k"][1],
                     grp[g]["k"][2], 1024, False)
        for g in range(3):
            rd[g].wait()
        for g in range(3):
            add_rows(g, recv3, 0, grp[g]["k"][2], 1024, True)

        for g in range(3):
            rd[g] = start_stage(g, 3, grp[g]["k"][2], 1024, None)
        for g in range(3):
            rd[g].wait()
        for g in range(3):
            rd[g] = start_stage(g, 4, grp[g]["k"][1], 2048, None)
        for g in range(3):
            rd[g].wait()
        for g in range(3):
            rd[g] = start_stage(g, 5, grp[g]["k"][0], 4096, None)
        for g in range(3):
            rd[g].wait()

    outs = pl.pallas_call(
        body,
        out_shape=(
            jax.ShapeDtypeStruct((M, N), jnp.float32),
            jax.ShapeDtypeStruct((4096, N), jnp.float32),
            jax.ShapeDtypeStruct((2048, N), jnp.float32),
            jax.ShapeDtypeStruct((1024, N), jnp.float32),
        ),
        in_specs=[
            pl.BlockSpec(memory_space=pl.ANY),
            pl.BlockSpec(memory_space=pltpu.VMEM),
        ],
        out_specs=(pl.BlockSpec(memory_space=pl.ANY),) * 4,
        scratch_shapes=[
            pltpu.VMEM((TM, K), jnp.float32),
            pltpu.VMEM((TM, WMAX), jnp.float32),
            pltpu.VMEM((TM, WMAX), jnp.float32),
            pltpu.VMEM((TM, WMAX), jnp.float32),
            pltpu.SemaphoreType.DMA((2,)),
            pltpu.SemaphoreType.DMA((3, 6)),
            pltpu.SemaphoreType.DMA((3, 6)),
        ],
        compiler_params=pltpu.CompilerParams(
            collective_id=0, vmem_limit_bytes=60 * 1024 * 1024),
    )(x, w_mat)
    return outs[0]


# device time: 1051510 ns/iter; 2.8097x vs baseline; 1.1301x over previous
import jax
import jax.numpy as jnp
from jax import lax
from jax.experimental import pallas as pl
from jax.experimental.pallas import tpu as pltpu

M = 8192
N = 4096
TM = 512
WIDTHS = (1408, 1408, 1280)
C0S = (0, 1408, 2816)
ROTS = ((0, 1, 2), (2, 0, 1), (1, 2, 0))
WMAX = 1408
S1A, S1B, S2A, S2B, S3A, S3B, SCA, SCB, SB, SA = range(10)


def kernel(x, w_mat):
    K = x.shape[1]

    def body(x_hbm, w_ref, out_hbm, recv1, recv2, recv3,
             x_vmem, gemm_vmem, acc_vmem, add_vmem,
             local_sems, ssems, rsems):
        my = lax.axis_index("i")
        z = my // 4
        j = my % 4
        y = j // 2
        xb = ((j + 1) // 2) % 2
        pz = my ^ 4
        py = 4 * z + (3 - j)
        px = my ^ 1
        axes = ((pz, z), (py, y), (px, xb))

        bsem = pltpu.get_barrier_semaphore()
        for p in (px, py, pz):
            pl.semaphore_signal(bsem, inc=1, device_id=(p,),
                                device_id_type=pl.DeviceIdType.MESH)
        pl.semaphore_wait(bsem, 3)

        grp = []
        for g in range(3):
            (p1, b1), (p2, b2), (p3, b3) = (axes[a] for a in ROTS[g])
            k1 = b1 * 4096
            k2 = k1 + b2 * 2048
            k3 = k2 + b3 * 1024
            s1 = (1 - b1) * 4096
            s2 = k1 + (1 - b2) * 2048
            s3 = k2 + (1 - b3) * 1024
            grp.append(dict(w=WIDTHS[g], c0=C0S[g], p=(p1, p2, p3),
                            b=(b1, b2, b3), k=(k1, k2, k3), s=(s1, s2, s3)))

        def cs(g):
            return pl.ds(grp[g]["c0"], grp[g]["w"])

        def gemm_rows(g, base, ntiles):
            w, c0 = grp[g]["w"], grp[g]["c0"]

            def step(i, c):
                r0 = base + i * TM
                ld = pltpu.make_async_copy(
                    x_hbm.at[pl.ds(r0, TM)], x_vmem, local_sems.at[0])
                ld.start()
                ld.wait()
                gemm_vmem[:, 0:w] = jnp.dot(
                    x_vmem[...], w_ref[:, c0:c0 + w],
                    preferred_element_type=jnp.float32)
                st = pltpu.make_async_copy(
                    gemm_vmem.at[:, pl.ds(0, w)],
                    out_hbm.at[pl.ds(r0, TM), cs(g)], local_sems.at[1])
                st.start()
                st.wait()
                return c

            lax.fori_loop(0, ntiles, step, 0)

        def add_rows(g, recv_buf, recv_r0, out_r0, nrows, fuse_silu):
            w = grp[g]["w"]

            def step(i, c):
                a = pltpu.make_async_copy(
                    out_hbm.at[pl.ds(out_r0 + i * TM, TM), cs(g)],
                    acc_vmem.at[:, pl.ds(0, w)], local_sems.at[0])
                b = pltpu.make_async_copy(
                    recv_buf.at[pl.ds(recv_r0 + i * TM, TM), cs(g)],
                    add_vmem.at[:, pl.ds(0, w)], local_sems.at[1])
                a.start()
                b.start()
                a.wait()
                b.wait()
                s = acc_vmem[:, 0:w] + add_vmem[:, 0:w]
                if fuse_silu:
                    s = s * jax.nn.sigmoid(s)
                acc_vmem[:, 0:w] = s
                st = pltpu.make_async_copy(
                    acc_vmem.at[:, pl.ds(0, w)],
                    out_hbm.at[pl.ds(out_r0 + i * TM, TM), cs(g)],
                    local_sems.at[0])
                st.start()
                st.wait()
                return c

            lax.fori_loop(0, nrows // TM, step, 0)

        def start(g, si, partner, src_base, nrows, recv_buf, dst_r0):
            src = out_hbm.at[pl.ds(src_base, nrows), cs(g)]
            if recv_buf is None:
                dst = src
            else:
                dst = recv_buf.at[pl.ds(dst_r0, nrows), cs(g)]
            r = pltpu.make_async_remote_copy(
                src_ref=src, dst_ref=dst,
                send_sem=ssems.at[g, si], recv_sem=rsems.at[g, si],
                device_id=(partner,), device_id_type=pl.DeviceIdType.MESH,
            )
            r.start()
            return r

        rd = [[None] * 10 for _ in range(3)]

        for g in range(3):
            G = grp[g]
            p1, b2 = G["p"][0], G["b"][1]
            suba = G["s"][0] + (1 - b2) * 2048
            subb = G["s"][0] + b2 * 2048
            gemm_rows(g, suba, 4)
            rd[g][S1A] = start(g, S1A, p1, suba, 2048, recv1, (1 - b2) * 2048)
            gemm_rows(g, subb, 4)
            rd[g][S1B] = start(g, S1B, p1, subb, 2048, recv1, b2 * 2048)
        for g in range(3):
            gemm_rows(g, grp[g]["k"][0], 8)

        for g in range(3):
            G = grp[g]
            p2, b3 = G["p"][1], G["b"][2]
            s2, k1 = G["s"][1], G["k"][0]
            rd[g][S1A].wait()
            add_rows(g, recv1, s2 - k1, s2, 2048, False)
            suba = s2 + (1 - b3) * 1024
            subb = s2 + b3 * 1024
            rd[g][S2A] = start(g, S2A, p2, suba, 1024, recv2, (1 - b3) * 1024)
            rd[g][S2B] = start(g, S2B, p2, subb, 1024, recv2, b3 * 1024)
        for g in range(3):
            G = grp[g]
            rd[g][S1B].wait()
            add_rows(g, recv1, G["k"][1] - G["k"][0], G["k"][1], 2048, False)
        for g in range(3):
            G = grp[g]
            p3, s3, k2 = G["p"][2], G["s"][2], G["k"][1]
            rd[g][S2A].wait()
            add_rows(g, recv2, s3 - k2, s3, 1024, False)
            rd[g][S3A] = start(g, S3A, p3, s3, 512, recv3, 0)
            rd[g][S3B] = start(g, S3B, p3, s3 + 512, 512, recv3, 512)
        for g in range(3):
            G = grp[g]
            rd[g][S2B].wait()
            add_rows(g, recv2, G["k"][2] - G["k"][1], G["k"][2], 1024, False)
        for g in range(3):
            G = grp[g]
            p3, k3 = G["p"][2], G["k"][2]
            rd[g][S3A].wait()
            add_rows(g, recv3, 0, k3, 512, True)
            rd[g][SCA] = start(g, SCA, p3, k3, 512, None, 0)
        for g in range(3):
            G = grp[g]
            p3, k3 = G["p"][2], G["k"][2]
            rd[g][S3B].wait()
            add_rows(g, recv3, 512, k3 + 512, 512, True)
            rd[g][SCB] = start(g, SCB, p3, k3 + 512, 512, None, 0)
        for g in range(3):
            G = grp[g]
            rd[g][SCA].wait()
            rd[g][SCB].wait()
            rd[g][SB] = start(g, SB, G["p"][1], G["k"][1], 2048, None, 0)
        for g in range(3):
            G = grp[g]
            rd[g][SB].wait()
            rd[g][SA] = start(g, SA, G["p"][0], G["k"][0], 4096, None, 0)
        for g in range(3):
            rd[g][SA].wait()

    outs = pl.pallas_call(
        body,
        out_shape=(
            jax.ShapeDtypeStruct((M, N), jnp.float32),
            jax.ShapeDtypeStruct((4096, N), jnp.float32),
            jax.ShapeDtypeStruct((2048, N), jnp.float32),
            jax.ShapeDtypeStruct((1024, N), jnp.float32),
        ),
        in_specs=[
            pl.BlockSpec(memory_space=pl.ANY),
            pl.BlockSpec(memory_space=pltpu.VMEM),
        ],
        out_specs=(pl.BlockSpec(memory_space=pl.ANY),) * 4,
        scratch_shapes=[
            pltpu.VMEM((TM, K), jnp.float32),
            pltpu.VMEM((TM, WMAX), jnp.float32),
            pltpu.VMEM((TM, WMAX), jnp.float32),
            pltpu.VMEM((TM, WMAX), jnp.float32),
            pltpu.SemaphoreType.DMA((2,)),
            pltpu.SemaphoreType.DMA((3, 10)),
            pltpu.SemaphoreType.DMA((3, 10)),
        ],
        compiler_params=pltpu.CompilerParams(
            collective_id=0, vmem_limit_bytes=60 * 1024 * 1024),
    )(x, w_mat)
    return outs[0]


# device time: 1032624 ns/iter; 2.8611x vs baseline; 1.0183x over previous
import jax
import jax.numpy as jnp
from jax import lax
from jax.experimental import pallas as pl
from jax.experimental.pallas import tpu as pltpu

M = 8192
N = 4096
TM = 512
WIDTHS = (1408, 1408, 1280)
C0S = (0, 1408, 2816)
ROTS = ((0, 1, 2), (2, 0, 1), (1, 2, 0))
WMAX = 1408
S1A, S1B, S2A, S2B, S3A, S3B, SCA, SCB, SBX, SBY, SA1, SA2, SA3 = range(13)


def kernel(x, w_mat):
    K = x.shape[1]

    def body(x_hbm, w_ref, out_hbm, recv1, recv2, recv3,
             x_vmem, gemm_vmem, acc_vmem, add_vmem,
             local_sems, ssems, rsems):
        my = lax.axis_index("i")
        z = my // 4
        j = my % 4
        y = j // 2
        xb = ((j + 1) // 2) % 2
        pz = my ^ 4
        py = 4 * z + (3 - j)
        px = my ^ 1
        axes = ((pz, z), (py, y), (px, xb))

        bsem = pltpu.get_barrier_semaphore()
        for p in (px, py, pz):
            pl.semaphore_signal(bsem, inc=1, device_id=(p,),
                                device_id_type=pl.DeviceIdType.MESH)
        pl.semaphore_wait(bsem, 3)

        grp = []
        for g in range(3):
            (p1, b1), (p2, b2), (p3, b3) = (axes[a] for a in ROTS[g])
            k1 = b1 * 4096
            k2 = k1 + b2 * 2048
            k3 = k2 + b3 * 1024
            s1 = (1 - b1) * 4096
            s2 = k1 + (1 - b2) * 2048
            s3 = k2 + (1 - b3) * 1024
            grp.append(dict(w=WIDTHS[g], c0=C0S[g], p=(p1, p2, p3),
                            b=(b1, b2, b3), k=(k1, k2, k3), s=(s1, s2, s3)))

        def cs(g):
            return pl.ds(grp[g]["c0"], grp[g]["w"])

        def gemm_rows(g, base, ntiles):
            w, c0 = grp[g]["w"], grp[g]["c0"]

            def step(i, c):
                r0 = base + i * TM
                ld = pltpu.make_async_copy(
                    x_hbm.at[pl.ds(r0, TM)], x_vmem, local_sems.at[0])
                ld.start()
                ld.wait()
                gemm_vmem[:, 0:w] = jnp.dot(
                    x_vmem[...], w_ref[:, c0:c0 + w],
                    preferred_element_type=jnp.float32)
                st = pltpu.make_async_copy(
                    gemm_vmem.at[:, pl.ds(0, w)],
                    out_hbm.at[pl.ds(r0, TM), cs(g)], local_sems.at[1])
                st.start()
                st.wait()
                return c

            lax.fori_loop(0, ntiles, step, 0)

        def add_rows(g, recv_buf, recv_r0, out_r0, nrows, fuse_silu):
            w = grp[g]["w"]

            def step(i, c):
                a = pltpu.make_async_copy(
                    out_hbm.at[pl.ds(out_r0 + i * TM, TM), cs(g)],
                    acc_vmem.at[:, pl.ds(0, w)], local_sems.at[0])
                b = pltpu.make_async_copy(
                    recv_buf.at[pl.ds(recv_r0 + i * TM, TM), cs(g)],
                    add_vmem.at[:, pl.ds(0, w)], local_sems.at[1])
                a.start()
                b.start()
                a.wait()
                b.wait()
                s = acc_vmem[:, 0:w] + add_vmem[:, 0:w]
                if fuse_silu:
                    s = s * jax.nn.sigmoid(s)
                acc_vmem[:, 0:w] = s
                st = pltpu.make_async_copy(
                    acc_vmem.at[:, pl.ds(0, w)],
                    out_hbm.at[pl.ds(out_r0 + i * TM, TM), cs(g)],
                    local_sems.at[0])
                st.start()
                st.wait()
                return c

            lax.fori_loop(0, nrows // TM, step, 0)

        def start(g, si, partner, src_base, nrows, recv_buf, dst_r0):
            src = out_hbm.at[pl.ds(src_base, nrows), cs(g)]
            if recv_buf is None:
                dst = src
            else:
                dst = recv_buf.at[pl.ds(dst_r0, nrows), cs(g)]
            r = pltpu.make_async_remote_copy(
                src_ref=src, dst_ref=dst,
                send_sem=ssems.at[g, si], recv_sem=rsems.at[g, si],
                device_id=(partner,), device_id_type=pl.DeviceIdType.MESH,
            )
            r.start()
            return r

        rd = [[None] * 13 for _ in range(3)]

        for g in range(3):
            G = grp[g]
            p1, b2 = G["p"][0], G["b"][1]
            suba = G["s"][0] + (1 - b2) * 2048
            subb = G["s"][0] + b2 * 2048
            gemm_rows(g, suba, 4)
            rd[g][S1A] = start(g, S1A, p1, suba, 2048, recv1, (1 - b2) * 2048)
            gemm_rows(g, subb, 4)
            rd[g][S1B] = start(g, S1B, p1, subb, 2048, recv1, b2 * 2048)
        for g in range(3):
            gemm_rows(g, grp[g]["k"][0], 8)

        for g in range(3):
            G = grp[g]
            p2, b3 = G["p"][1], G["b"][2]
            s2, k1 = G["s"][1], G["k"][0]
            rd[g][S1A].wait()
            add_rows(g, recv1, s2 - k1, s2, 2048, False)
            suba = s2 + (1 - b3) * 1024
            subb = s2 + b3 * 1024
            rd[g][S2A] = start(g, S2A, p2, suba, 1024, recv2, (1 - b3) * 1024)
            rd[g][S2B] = start(g, S2B, p2, subb, 1024, recv2, b3 * 1024)
        for g in range(3):
            G = grp[g]
            rd[g][S1B].wait()
            add_rows(g, recv1, G["k"][1] - G["k"][0], G["k"][1], 2048, False)
        for g in range(3):
            G = grp[g]
            p3, s3, k2 = G["p"][2], G["s"][2], G["k"][1]
            rd[g][S2A].wait()
            add_rows(g, recv2, s3 - k2, s3, 1024, False)
            rd[g][S3A] = start(g, S3A, p3, s3, 512, recv3, 0)
            rd[g][S3B] = start(g, S3B, p3, s3 + 512, 512, recv3, 512)
        for g in range(3):
            G = grp[g]
            rd[g][S2B].wait()
            add_rows(g, recv2, G["k"][2] - G["k"][1], G["k"][2], 1024, False)
        for g in range(3):
            G = grp[g]
            p3, k3 = G["p"][2], G["k"][2]
            rd[g][S3A].wait()
            add_rows(g, recv3, 0, k3, 512, True)
            rd[g][SCA] = start(g, SCA, p3, k3, 512, None, 0)
        for g in range(3):
            G = grp[g]
            p1, p2, p3, k3 = G["p"][0], G["p"][1], G["p"][2], G["k"][2]
            rd[g][S3B].wait()
            add_rows(g, recv3, 512, k3 + 512, 512, True)
            rd[g][SCB] = start(g, SCB, p3, k3 + 512, 512, None, 0)
            rd[g][SBX] = start(g, SBX, p2, k3, 1024, None, 0)
            rd[g][SA1] = start(g, SA1, p1, k3, 1024, None, 0)
        for g in range(3):
            G = grp[g]
            rd[g][SCA].wait()
            rd[g][SCB].wait()
            rd[g][SBY] = start(g, SBY, G["p"][1], G["s"][2], 1024, None, 0)
            rd[g][SA2] = start(g, SA2, G["p"][0], G["s"][2], 1024, None, 0)
        for g in range(3):
            G = grp[g]
            rd[g][SBX].wait()
            rd[g][SBY].wait()
            rd[g][SA3] = start(g, SA3, G["p"][0], G["s"][1], 2048, None, 0)
        for g in range(3):
            rd[g][SA1].wait()
            rd[g][SA2].wait()
            rd[g][SA3].wait()

    outs = pl.pallas_call(
        body,
        out_shape=(
            jax.ShapeDtypeStruct((M, N), jnp.float32),
            jax.ShapeDtypeStruct((4096, N), jnp.float32),
            jax.ShapeDtypeStruct((2048, N), jnp.float32),
            jax.ShapeDtypeStruct((1024, N), jnp.float32),
        ),
        in_specs=[
            pl.BlockSpec(memory_space=pl.ANY),
            pl.BlockSpec(memory_space=pltpu.VMEM),
        ],
        out_specs=(pl.BlockSpec(memory_space=pl.ANY),) * 4,
        scratch_shapes=[
            pltpu.VMEM((TM, K), jnp.float32),
            pltpu.VMEM((TM, WMAX), jnp.float32),
            pltpu.VMEM((TM, WMAX), jnp.float32),
            pltpu.VMEM((TM, WMAX), jnp.float32),
            pltpu.SemaphoreType.DMA((2,)),
            pltpu.SemaphoreType.DMA((3, 13)),
            pltpu.SemaphoreType.DMA((3, 13)),
        ],
        compiler_params=pltpu.CompilerParams(
            collective_id=0, vmem_limit_bytes=60 * 1024 * 1024),
    )(x, w_mat)
    return outs[0]


# device time: 1002072 ns/iter; 2.9483x vs baseline; 1.0305x over previous
import jax
import jax.numpy as jnp
from jax import lax
from jax.experimental import pallas as pl
from jax.experimental.pallas import tpu as pltpu

M = 8192
N = 4096
TM = 512
WIDTHS = (1408, 1408, 1280)
C0S = (0, 1408, 2816)
ROTS = ((0, 1, 2), (2, 0, 1), (1, 2, 0))
WMAX = 1408
S1A, S1B, S2A, S2B, S3A, S3B, SCA, SCB, SBX, SBY, SA1, SA2, SA3 = range(13)


def kernel(x, w_mat):
    K = x.shape[1]

    def body(x_hbm, w_ref, out_hbm, recv1, recv2, recv3,
             x_vmem, gemm_vmem, acc_vmem, add_vmem,
             local_sems, ssems, rsems):
        my = lax.axis_index("i")
        z = my // 4
        j = my % 4
        y = j // 2
        xb = ((j + 1) // 2) % 2
        pz = my ^ 4
        py = 4 * z + (3 - j)
        px = my ^ 1
        axes = ((pz, z), (py, y), (px, xb))

        bsem = pltpu.get_barrier_semaphore()
        for p in (px, py, pz):
            pl.semaphore_signal(bsem, inc=1, device_id=(p,),
                                device_id_type=pl.DeviceIdType.MESH)
        pl.semaphore_wait(bsem, 3)

        grp = []
        for g in range(3):
            (p1, b1), (p2, b2), (p3, b3) = (axes[a] for a in ROTS[g])
            k1 = b1 * 4096
            k2 = k1 + b2 * 2048
            k3 = k2 + b3 * 1024
            s1 = (1 - b1) * 4096
            s2 = k1 + (1 - b2) * 2048
            s3 = k2 + (1 - b3) * 1024
            grp.append(dict(w=WIDTHS[g], c0=C0S[g], p=(p1, p2, p3),
                            b=(b1, b2, b3), k=(k1, k2, k3), s=(s1, s2, s3)))

        def cs(g):
            return pl.ds(grp[g]["c0"], grp[g]["w"])

        def gemm_rows(g, base, ntiles):
            w, c0 = grp[g]["w"], grp[g]["c0"]

            def step(i, c):
                r0 = base + i * TM
                ld = pltpu.make_async_copy(
                    x_hbm.at[pl.ds(r0, TM)], x_vmem, local_sems.at[0])
                ld.start()
                ld.wait()
                gemm_vmem[:, 0:w] = jnp.dot(
                    x_vmem[...], w_ref[:, c0:c0 + w],
                    preferred_element_type=jnp.float32)
                st = pltpu.make_async_copy(
                    gemm_vmem.at[:, pl.ds(0, w)],
                    out_hbm.at[pl.ds(r0, TM), cs(g)], local_sems.at[1])
                st.start()
                st.wait()
                return c

            lax.fori_loop(0, ntiles, step, 0)

        def add_rows(g, recv_buf, recv_r0, out_r0, nrows, fuse_silu):
            w = grp[g]["w"]

            def step(i, c):
                a = pltpu.make_async_copy(
                    out_hbm.at[pl.ds(out_r0 + i * TM, TM), cs(g)],
                    acc_vmem.at[:, pl.ds(0, w)], local_sems.at[0])
                b = pltpu.make_async_copy(
                    recv_buf.at[pl.ds(recv_r0 + i * TM, TM), cs(g)],
                    add_vmem.at[:, pl.ds(0, w)], local_sems.at[1])
                a.start()
                b.start()
                a.wait()
                b.wait()
                s = acc_vmem[:, 0:w] + add_vmem[:, 0:w]
                if fuse_silu:
                    s = s * jax.nn.sigmoid(s)
                acc_vmem[:, 0:w] = s
                st = pltpu.make_async_copy(
                    acc_vmem.at[:, pl.ds(0, w)],
                    out_hbm.at[pl.ds(out_r0 + i * TM, TM), cs(g)],
                    local_sems.at[0])
                st.start()
                st.wait()
                return c

            lax.fori_loop(0, nrows // TM, step, 0)

        def start(g, si, partner, src_base, nrows, recv_buf, dst_r0):
            src = out_hbm.at[pl.ds(src_base, nrows), cs(g)]
            if recv_buf is None:
                dst = src
            else:
                dst = recv_buf.at[pl.ds(dst_r0, nrows), cs(g)]
            r = pltpu.make_async_remote_copy(
                src_ref=src, dst_ref=dst,
                send_sem=ssems.at[g, si], recv_sem=rsems.at[g, si],
                device_id=(partner,), device_id_type=pl.DeviceIdType.MESH,
            )
            r.start()
            return r

        rd = [[None] * 13 for _ in range(3)]

        for g in range(3):
            G = grp[g]
            p1, b2 = G["p"][0], G["b"][1]
            suba = G["s"][0] + (1 - b2) * 2048
            gemm_rows(g, suba, 4)
            rd[g][S1A] = start(g, S1A, p1, suba, 2048, recv1, (1 - b2) * 2048)
        for g in range(3):
            G = grp[g]
            p1, b2 = G["p"][0], G["b"][1]
            subb = G["s"][0] + b2 * 2048
            gemm_rows(g, subb, 4)
            rd[g][S1B] = start(g, S1B, p1, subb, 2048, recv1, b2 * 2048)
        for g in range(3):
            gemm_rows(g, grp[g]["k"][0], 8)

        for g in range(3):
            G = grp[g]
            p2, b3 = G["p"][1], G["b"][2]
            s2, k1 = G["s"][1], G["k"][0]
            rd[g][S1A].wait()
            suba = s2 + (1 - b3) * 1024
            subb = s2 + b3 * 1024
            add_rows(g, recv1, suba - k1, suba, 1024, False)
            rd[g][S2A] = start(g, S2A, p2, suba, 1024, recv2, (1 - b3) * 1024)
            add_rows(g, recv1, subb - k1, subb, 1024, False)
            rd[g][S2B] = start(g, S2B, p2, subb, 1024, recv2, b3 * 1024)
        for g in range(3):
            G = grp[g]
            rd[g][S1B].wait()
            add_rows(g, recv1, G["k"][1] - G["k"][0], G["k"][1], 2048, False)
        for g in range(3):
            G = grp[g]
            p3, s3, k2 = G["p"][2], G["s"][2], G["k"][1]
            rd[g][S2A].wait()
            add_rows(g, recv2, s3 - k2, s3, 512, False)
            rd[g][S3A] = start(g, S3A, p3, s3, 512, recv3, 0)
            add_rows(g, recv2, s3 + 512 - k2, s3 + 512, 512, False)
            rd[g][S3B] = start(g, S3B, p3, s3 + 512, 512, recv3, 512)
        for g in range(3):
            G = grp[g]
            rd[g][S2B].wait()
            add_rows(g, recv2, G["k"][2] - G["k"][1], G["k"][2], 1024, False)
        for g in range(3):
            G = grp[g]
            p3, k3 = G["p"][2], G["k"][2]
            rd[g][S3A].wait()
            add_rows(g, recv3, 0, k3, 512, True)
            rd[g][SCA] = start(g, SCA, p3, k3, 512, None, 0)
        for g in range(3):
            G = grp[g]
            p1, p2, p3, k3 = G["p"][0], G["p"][1], G["p"][2], G["k"][2]
            rd[g][S3B].wait()
            add_rows(g, recv3, 512, k3 + 512, 512, True)
            rd[g][SCB] = start(g, SCB, p3, k3 + 512, 512, None, 0)
            rd[g][SBX] = start(g, SBX, p2, k3, 1024, None, 0)
            rd[g][SA1] = start(g, SA1, p1, k3, 1024, None, 0)
        for g in range(3):
            G = grp[g]
            rd[g][SCA].wait()
            rd[g][SCB].wait()
            rd[g][SBY] = start(g, SBY, G["p"][1], G["s"][2], 1024, None, 0)
            rd[g][SA2] = start(g, SA2, G["p"][0], G["s"][2], 1024, None, 0)
        for g in range(3):
            G = grp[g]
            rd[g][SBX].wait()
            rd[g][SBY].wait()
            rd[g][SA3] = start(g, SA3, G["p"][0], G["s"][1], 2048, None, 0)
        for g in range(3):
            rd[g][SA1].wait()
            rd[g][SA2].wait()
            rd[g][SA3].wait()

    outs = pl.pallas_call(
        body,
        out_shape=(
            jax.ShapeDtypeStruct((M, N), jnp.float32),
            jax.ShapeDtypeStruct((4096, N), jnp.float32),
            jax.ShapeDtypeStruct((2048, N), jnp.float32),
            jax.ShapeDtypeStruct((1024, N), jnp.float32),
        ),
        in_specs=[
            pl.BlockSpec(memory_space=pl.ANY),
            pl.BlockSpec(memory_space=pltpu.VMEM),
        ],
        out_specs=(pl.BlockSpec(memory_space=pl.ANY),) * 4,
        scratch_shapes=[
            pltpu.VMEM((TM, K), jnp.float32),
            pltpu.VMEM((TM, WMAX), jnp.float32),
            pltpu.VMEM((TM, WMAX), jnp.float32),
            pltpu.VMEM((TM, WMAX), jnp.float32),
            pltpu.SemaphoreType.DMA((2,)),
            pltpu.SemaphoreType.DMA((3, 13)),
            pltpu.SemaphoreType.DMA((3, 13)),
        ],
        compiler_params=pltpu.CompilerParams(
            collective_id=0, vmem_limit_bytes=60 * 1024 * 1024),
    )(x, w_mat)
    return outs[0]
